# Initial kernel scaffold; baseline (speedup 1.0000x reference)
import jax
import jax.numpy as jnp
from jax.experimental import pallas as pl


def kernel(x1, x2, edge_index1, edge_index2, edge_attr1, edge_attr2, W1, b1, W4, b4, Wc1, bc1, Wc2, bc2):
    raise NotImplementedError("write your pallas kernel here")


# speedup vs baseline: 151.8676x; 151.8676x over previous
import functools

import jax
import jax.numpy as jnp
from jax import lax
from jax.experimental import pallas as pl
from jax.experimental.pallas import tpu as pltpu
from jax.experimental.pallas import tpu_sc as plsc

NN = 268
EE = 8576
BB = 32
GS = 2 * BB
NP = 272
NF = 128
NH = 64
EPS = 1e-6

_NC = 2
_NS = 16
_NW = _NC * _NS
_SIDES_PER_W = GS // _NW
_LANES = 16
_WCELLS = NP * NP


def _sc_body(row_hbm, col_hbm, ew_hbm, zeros_hbm, out_hbm,
             row_v, col_v, ew_v, w_v):
  wid = lax.axis_index("s") * _NC + lax.axis_index("c")
  for g in range(_SIDES_PER_W):
    side = wid * _SIDES_PER_W + g
    pltpu.sync_copy(zeros_hbm, w_v)
    pltpu.sync_copy(row_hbm.at[side], row_v)
    pltpu.sync_copy(col_hbm.at[side], col_v)
    pltpu.sync_copy(ew_hbm.at[side], ew_v)

    def ebody(i, carry):
      base = i * _LANES
      r = row_v[pl.ds(base, _LANES)]
      c = col_v[pl.ds(base, _LANES)]
      w = ew_v[pl.ds(base, _LANES)]
      idx = c * NP + r
      plsc.addupdate_scatter(w_v, [idx], w)
      return carry

    lax.fori_loop(0, EE // _LANES, ebody, 0)
    pltpu.sync_copy(w_v, out_hbm.at[side])


@functools.cache
def _sc_build():
  return pl.kernel(
      _sc_body,
      out_type=jax.ShapeDtypeStruct((GS, _WCELLS), jnp.float32),
      mesh=plsc.VectorSubcoreMesh(core_axis_name="c", subcore_axis_name="s"),
      scratch_types=[
          pltpu.VMEM((EE,), jnp.int32),
          pltpu.VMEM((EE,), jnp.int32),
          pltpu.VMEM((EE,), jnp.float32),
          pltpu.VMEM((_WCELLS,), jnp.float32),
      ],
      compiler_params=pltpu.CompilerParams(needs_layout_passes=False),
  )


def _tc_body(wd_ref, x_ref, w1_ref, b1_ref, w4_ref, b4_ref,
             wc1_ref, bc1_ref, wc2_ref, bc2_ref, out_ref):
  dot = functools.partial(
      lax.dot_general,
      dimension_numbers=(((1,), (0,)), ((), ())),
      precision=lax.Precision.HIGHEST,
      preferred_element_type=jnp.float32,
  )
  b1 = b1_ref[...]
  b4 = b4_ref[0]
  os = []
  for s in range(2):
    wd = wd_ref[0, s]
    x = x_ref[0, s]
    deg = jnp.sum(wd, axis=0)
    dinv = jnp.where(deg > 0, lax.rsqrt(jnp.where(deg > 0, deg, 1.0)), 0.0)
    a = -(dinv[:, None] * wd * dinv[None, :])

    tx1 = dot(a, x)
    tx2 = 2.0 * dot(a, tx1) - x
    h = dot(x, w1_ref[0]) + dot(tx1, w1_ref[1]) + dot(tx2, w1_ref[2])
    h = jnp.maximum(h + b1[None, :], 0.0)

    th1 = dot(a, h)
    th2 = 2.0 * dot(a, th1) - h
    w4 = w4_ref[...]
    o = (jnp.sum(h * w4[0, :, 0][None, :], axis=1)
         + jnp.sum(th1 * w4[1, :, 0][None, :], axis=1)
         + jnp.sum(th2 * w4[2, :, 0][None, :], axis=1) + b4)
    os.append(o)

  dist = jnp.abs(os[0] - os[1] + EPS)
  cls = jnp.sum(dist[:, None] * wc1_ref[...], axis=0)
  hc = jnp.maximum(cls + bc1_ref[...], 0.0)
  res = jnp.sum(hc * wc2_ref[...][:, 0]) + bc2_ref[0]
  out_ref[0] = res[None, None]


def _tc_pairs(wd, xs, W1, b1, W4, b4, Wc1p, bc1p, Wc2p, bc2):
  full = lambda shape: pl.BlockSpec(shape, lambda b: (0,) * len(shape))
  return pl.pallas_call(
      _tc_body,
      grid=(BB,),
      in_specs=[
          pl.BlockSpec((1, 2, NP, NP), lambda b: (b, 0, 0, 0)),
          pl.BlockSpec((1, 2, NP, NF), lambda b: (b, 0, 0, 0)),
          full((3, NF, NH)),
          full((NH,)),
          full((3, NH, 1)),
          full((1,)),
          full((NP, NH)),
          full((NH,)),
          full((NH, 1)),
          full((1,)),
      ],
      out_specs=pl.BlockSpec((1, 1, 1), lambda b: (b, 0, 0)),
      out_shape=jax.ShapeDtypeStruct((BB, 1, 1), jnp.float32),
  )(wd, xs, W1, b1, W4, b4, Wc1p, bc1p, Wc2p, bc2).reshape(BB, 1)


def kernel(x1, x2, edge_index1, edge_index2, edge_attr1, edge_attr2,
           W1, b1, W4, b4, Wc1, bc1, Wc2, bc2):
  row = jnp.stack([edge_index1[:, 0], edge_index2[:, 0]], 1).reshape(GS, EE)
  col = jnp.stack([edge_index1[:, 1], edge_index2[:, 1]], 1).reshape(GS, EE)
  ew = jnp.stack([edge_attr1, edge_attr2], 1).reshape(GS, EE)
  zeros = jnp.zeros((_WCELLS,), jnp.float32)

  wd = _sc_build()(row, col, ew, zeros).reshape(BB, 2, NP, NP)

  xs = jnp.stack([x1, x2], axis=1)
  xs = jnp.pad(xs, ((0, 0), (0, 0), (0, NP - NN), (0, 0)))
  Wc1p = jnp.pad(Wc1, ((0, NP - NN), (0, NH - 60)))
  bc1p = jnp.pad(bc1, (0, NH - 60))
  Wc2p = jnp.pad(Wc2, ((0, NH - 60), (0, 0)))

  return _tc_pairs(wd, xs, W1, b1, W4, b4, Wc1p, bc1p, Wc2p, bc2)


# speedup vs baseline: 154.7639x; 1.0191x over previous
import functools

import jax
import jax.numpy as jnp
from jax import lax
from jax.experimental import pallas as pl
from jax.experimental.pallas import tpu as pltpu
from jax.experimental.pallas import tpu_sc as plsc

NN = 268
EE = 8576
BB = 32
GS = 2 * BB
NP = 272
NF = 128
NH = 64
EPS = 1e-6

_NC = 2
_NS = 16
_NW = _NC * _NS
_SIDES_PER_W = GS // _NW
_LANES = 16
_WCELLS = NP * NP


def _sc_body(ei1_hbm, ei2_hbm, ea1_hbm, ea2_hbm, zeros_hbm, out_hbm,
             row_v, col_v, ew_v, w_v):
  wid = lax.axis_index("s") * _NC + lax.axis_index("c")
  for g, (ei_hbm, ea_hbm) in enumerate(((ei1_hbm, ea1_hbm),
                                        (ei2_hbm, ea2_hbm))):
    pltpu.sync_copy(zeros_hbm, w_v)
    pltpu.sync_copy(ei_hbm.at[wid, 0], row_v)
    pltpu.sync_copy(ei_hbm.at[wid, 1], col_v)
    pltpu.sync_copy(ea_hbm.at[wid], ew_v)

    def ebody(i, carry):
      base = i * _LANES
      r = row_v[pl.ds(base, _LANES)]
      c = col_v[pl.ds(base, _LANES)]
      w = ew_v[pl.ds(base, _LANES)]
      idx = c * NP + r
      plsc.addupdate_scatter(w_v, [idx], w)
      return carry

    lax.fori_loop(0, EE // _LANES, ebody, 0)
    pltpu.sync_copy(w_v, out_hbm.at[wid, g])


@functools.cache
def _sc_build():
  return pl.kernel(
      _sc_body,
      out_type=jax.ShapeDtypeStruct((BB, 2, _WCELLS), jnp.float32),
      mesh=plsc.VectorSubcoreMesh(core_axis_name="c", subcore_axis_name="s"),
      scratch_types=[
          pltpu.VMEM((EE,), jnp.int32),
          pltpu.VMEM((EE,), jnp.int32),
          pltpu.VMEM((EE,), jnp.float32),
          pltpu.VMEM((_WCELLS,), jnp.float32),
      ],
      compiler_params=pltpu.CompilerParams(needs_layout_passes=False),
  )


def _tc_body(wd_ref, x1_ref, x2_ref, w1_ref, b1_ref, w4_ref, b4_ref,
             wc1_ref, bc1_ref, wc2_ref, bc2_ref, out_ref):
  dot = functools.partial(
      lax.dot_general,
      dimension_numbers=(((1,), (0,)), ((), ())),
      precision=lax.Precision.HIGHEST,
      preferred_element_type=jnp.float32,
  )
  b1 = b1_ref[...]
  b4 = b4_ref[0]
  os = []
  for s, x_ref in enumerate((x1_ref, x2_ref)):
    wd = wd_ref[0, s]
    x = jnp.concatenate(
        [x_ref[0], jnp.zeros((NP - NN, NF), jnp.float32)], axis=0)
    deg = jnp.sum(wd, axis=0)
    dinv = jnp.where(deg > 0, lax.rsqrt(jnp.where(deg > 0, deg, 1.0)), 0.0)
    a = -(dinv[:, None] * wd * dinv[None, :])

    tx1 = dot(a, x)
    tx2 = 2.0 * dot(a, tx1) - x
    h = dot(x, w1_ref[0]) + dot(tx1, w1_ref[1]) + dot(tx2, w1_ref[2])
    h = jnp.maximum(h + b1[None, :], 0.0)

    th1 = dot(a, h)
    th2 = 2.0 * dot(a, th1) - h
    w4 = w4_ref[...]
    o = (jnp.sum(h * w4[0, :, 0][None, :], axis=1)
         + jnp.sum(th1 * w4[1, :, 0][None, :], axis=1)
         + jnp.sum(th2 * w4[2, :, 0][None, :], axis=1) + b4)
    os.append(o)

  dist = jnp.abs(os[0] - os[1] + EPS)
  cls = jnp.sum(dist[:, None] * wc1_ref[...], axis=0)
  hc = jnp.maximum(cls + bc1_ref[...], 0.0)
  res = jnp.sum(hc * wc2_ref[...][:, 0]) + bc2_ref[0]
  out_ref[0] = res[None, None]


def _tc_pairs(wd, x1, x2, W1, b1, W4, b4, Wc1p, bc1p, Wc2p, bc2):
  full = lambda shape: pl.BlockSpec(shape, lambda b: (0,) * len(shape))
  return pl.pallas_call(
      _tc_body,
      grid=(BB,),
      in_specs=[
          pl.BlockSpec((1, 2, NP, NP), lambda b: (b, 0, 0, 0)),
          pl.BlockSpec((1, NN, NF), lambda b: (b, 0, 0)),
          pl.BlockSpec((1, NN, NF), lambda b: (b, 0, 0)),
          full((3, NF, NH)),
          full((NH,)),
          full((3, NH, 1)),
          full((1,)),
          full((NP, NH)),
          full((NH,)),
          full((NH, 1)),
          full((1,)),
      ],
      out_specs=pl.BlockSpec((1, 1, 1), lambda b: (b, 0, 0)),
      out_shape=jax.ShapeDtypeStruct((BB, 1, 1), jnp.float32),
  )(wd, x1, x2, W1, b1, W4, b4, Wc1p, bc1p, Wc2p, bc2).reshape(BB, 1)


def kernel(x1, x2, edge_index1, edge_index2, edge_attr1, edge_attr2,
           W1, b1, W4, b4, Wc1, bc1, Wc2, bc2):
  zeros = jnp.zeros((_WCELLS,), jnp.float32)
  wd = _sc_build()(edge_index1, edge_index2, edge_attr1, edge_attr2,
                   zeros).reshape(BB, 2, NP, NP)

  Wc1p = jnp.pad(Wc1, ((0, NP - NN), (0, NH - 60)))
  bc1p = jnp.pad(bc1, (0, NH - 60))
  Wc2p = jnp.pad(Wc2, ((0, NH - 60), (0, 0)))

  return _tc_pairs(wd, x1, x2, W1, b1, W4, b4, Wc1p, bc1p, Wc2p, bc2)


# speedup vs baseline: 267.0267x; 1.7254x over previous
import functools

import jax
import jax.numpy as jnp
from jax import lax
from jax.experimental import pallas as pl
from jax.experimental.pallas import tpu as pltpu
from jax.experimental.pallas import tpu_sc as plsc

NN = 268
EE = 8576
BB = 32
GS = 2 * BB
NP = 272
NF = 128
NH = 64
EPS = 1e-6

_NC = 2
_NS = 16
_NW = _NC * _NS
_SIDES_PER_W = GS // _NW
_LANES = 16
_WCELLS = NP * NP


def _sc_body(ei1_hbm, ei2_hbm, ea1_hbm, ea2_hbm, zeros_hbm, out_hbm,
             row_v, col_v, ew_v, w_v):
  wid = lax.axis_index("s") * _NC + lax.axis_index("c")
  for g, (ei_hbm, ea_hbm) in enumerate(((ei1_hbm, ea1_hbm),
                                        (ei2_hbm, ea2_hbm))):
    pltpu.sync_copy(zeros_hbm, w_v)
    pltpu.sync_copy(ei_hbm.at[wid, 0], row_v)
    pltpu.sync_copy(ei_hbm.at[wid, 1], col_v)
    pltpu.sync_copy(ea_hbm.at[wid], ew_v)

    def ebody(i, carry):
      base = i * _LANES
      r = row_v[pl.ds(base, _LANES)]
      c = col_v[pl.ds(base, _LANES)]
      w = ew_v[pl.ds(base, _LANES)]
      idx = c * NP + r
      plsc.addupdate_scatter(w_v, [idx], w)
      return carry

    lax.fori_loop(0, EE // _LANES, ebody, 0)
    pltpu.sync_copy(w_v, out_hbm.at[wid, g])


@functools.cache
def _sc_build():
  return pl.kernel(
      _sc_body,
      out_type=jax.ShapeDtypeStruct((BB, 2, _WCELLS), jnp.float32),
      mesh=plsc.VectorSubcoreMesh(core_axis_name="c", subcore_axis_name="s"),
      scratch_types=[
          pltpu.VMEM((EE,), jnp.int32),
          pltpu.VMEM((EE,), jnp.int32),
          pltpu.VMEM((EE,), jnp.float32),
          pltpu.VMEM((_WCELLS,), jnp.float32),
      ],
      compiler_params=pltpu.CompilerParams(needs_layout_passes=False),
  )


def _tc_body(wd_ref, x1_ref, x2_ref, w1_ref, b1_ref, w4_ref, b4_ref,
             wc1_ref, bc1_ref, wc2_ref, bc2_ref, out_ref):
  dot = functools.partial(
      lax.dot_general,
      dimension_numbers=(((1,), (0,)), ((), ())),
      precision=lax.Precision.DEFAULT,
      preferred_element_type=jnp.float32,
  )
  b1 = b1_ref[...]
  b4 = b4_ref[0]
  os = []
  for s, x_ref in enumerate((x1_ref, x2_ref)):
    wd = wd_ref[0, s]
    x = jnp.concatenate(
        [x_ref[0], jnp.zeros((NP - NN, NF), jnp.float32)], axis=0)
    deg = jnp.sum(wd, axis=0)
    dinv = jnp.where(deg > 0, lax.rsqrt(jnp.where(deg > 0, deg, 1.0)), 0.0)
    a = -(dinv[:, None] * wd * dinv[None, :])

    tx1 = dot(a, x)
    tx2 = 2.0 * dot(a, tx1) - x
    h = dot(x, w1_ref[0]) + dot(tx1, w1_ref[1]) + dot(tx2, w1_ref[2])
    h = jnp.maximum(h + b1[None, :], 0.0)

    th1 = dot(a, h)
    th2 = 2.0 * dot(a, th1) - h
    w4 = w4_ref[...]
    o = (jnp.sum(h * w4[0, :, 0][None, :], axis=1)
         + jnp.sum(th1 * w4[1, :, 0][None, :], axis=1)
         + jnp.sum(th2 * w4[2, :, 0][None, :], axis=1) + b4)
    os.append(o)

  dist = jnp.abs(os[0] - os[1] + EPS)
  cls = jnp.sum(dist[:, None] * wc1_ref[...], axis=0)
  hc = jnp.maximum(cls + bc1_ref[...], 0.0)
  res = jnp.sum(hc * wc2_ref[...][:, 0]) + bc2_ref[0]
  out_ref[0] = res[None, None]


def _tc_pairs(wd, x1, x2, W1, b1, W4, b4, Wc1p, bc1p, Wc2p, bc2):
  full = lambda shape: pl.BlockSpec(shape, lambda b: (0,) * len(shape))
  return pl.pallas_call(
      _tc_body,
      grid=(BB,),
      in_specs=[
          pl.BlockSpec((1, 2, NP, NP), lambda b: (b, 0, 0, 0)),
          pl.BlockSpec((1, NN, NF), lambda b: (b, 0, 0)),
          pl.BlockSpec((1, NN, NF), lambda b: (b, 0, 0)),
          full((3, NF, NH)),
          full((NH,)),
          full((3, NH, 1)),
          full((1,)),
          full((NP, NH)),
          full((NH,)),
          full((NH, 1)),
          full((1,)),
      ],
      out_specs=pl.BlockSpec((1, 1, 1), lambda b: (b, 0, 0)),
      out_shape=jax.ShapeDtypeStruct((BB, 1, 1), jnp.float32),
  )(wd, x1, x2, W1, b1, W4, b4, Wc1p, bc1p, Wc2p, bc2).reshape(BB, 1)


def kernel(x1, x2, edge_index1, edge_index2, edge_attr1, edge_attr2,
           W1, b1, W4, b4, Wc1, bc1, Wc2, bc2):
  zeros = jnp.zeros((_WCELLS,), jnp.float32)
  wd = _sc_build()(edge_index1, edge_index2, edge_attr1, edge_attr2,
                   zeros).reshape(BB, 2, NP, NP)

  Wc1p = jnp.pad(Wc1, ((0, NP - NN), (0, NH - 60)))
  bc1p = jnp.pad(bc1, (0, NH - 60))
  Wc2p = jnp.pad(Wc2, ((0, NH - 60), (0, 0)))

  return _tc_pairs(wd, x1, x2, W1, b1, W4, b4, Wc1p, bc1p, Wc2p, bc2)
